# baseline (speedup 1.0000x reference)
import functools
import math

import jax
import jax.numpy as jnp
from jax import lax
from jax.experimental import pallas as pl
from jax.experimental.pallas import tpu as pltpu
from jax.experimental.pallas import tpu_sc as plsc

B = 1024
V = 100000
SMOOTH = 0.1 / (V - 2)
CONF = 0.9
Z_COL = V - 100
A_CONST = SMOOTH * math.log(SMOOTH) * (V - 2) + CONF * math.log(CONF)
SLOGS = SMOOTH * math.log(SMOOTH)

NW = 32
CR = 32
RPW = 1568
V_SC = NW * RPW
NCH = RPW // CR
BPW = B // NW
UNROLL = 8
NACC = 8
NVEC_ROW = B // 16

TC_RB = 1024
TC_R0 = V_SC // TC_RB
TC_NR = (V - V_SC + TC_RB - 1) // TC_RB


def _sc_body(outT_hbm, tgt_hbm, res_hbm, buf0, buf1, tbuf, zbuf, tgt_v,
             out_v, sem0, sem1, semg):
    wid = lax.axis_index("s") * 2 + lax.axis_index("c")
    v0 = wid * RPW
    b0 = wid * BPW
    pltpu.sync_copy(tgt_hbm.at[pl.ds(b0, BPW)], tgt_v)
    iota = lax.iota(jnp.int32, 16)
    lane0 = jnp.where(iota == 0, jnp.float32(1.0), jnp.float32(0.0))
    zero = jnp.zeros((16,), jnp.float32)

    def get_t(i):
        return tgt_v[pl.ds((i // 16) * 16, 16)][i % 16]

    bcol = pl.multiple_of((b0 & ~127), 128)
    pltpu.async_copy(
        outT_hbm.at[pl.ds((Z_COL // 8) * 8, 8), pl.ds(bcol, 128)], zbuf, semg)

    def dma_chunk(j, buf, sem):
        return pltpu.async_copy(
            outT_hbm.at[pl.ds(v0 + j * CR, CR), pl.ds(0, B)], buf, sem)

    def sum_chunk(buf, accs):
        def row_body(rr, accs):
            def body(ii, accs):
                a = list(accs)
                base = ii * (UNROLL * 16)
                for u in range(UNROLL):
                    a[u] = a[u] + buf[rr, pl.ds(base + u * 16, 16)]
                return tuple(a)
            return lax.fori_loop(0, NVEC_ROW // UNROLL, body, accs)
        return lax.fori_loop(0, CR, row_body, accs)

    dma_chunk(0, buf0, sem0)
    accs = (zero,) * NACC

    def pair_body(k, accs):
        j = 2 * k
        pltpu.make_async_copy(
            outT_hbm.at[pl.ds(0, CR), pl.ds(0, B)], buf0, sem0).wait()
        dma_chunk(j + 1, buf1, sem1)
        accs = sum_chunk(buf0, accs)
        dma_chunk(j + 2, buf0, sem0)
        pltpu.make_async_copy(
            outT_hbm.at[pl.ds(0, CR), pl.ds(0, B)], buf1, sem1).wait()
        accs = sum_chunk(buf1, accs)
        return accs

    accs = lax.fori_loop(0, (NCH - 1) // 2, pair_body, accs)
    pltpu.make_async_copy(
        outT_hbm.at[pl.ds(0, CR), pl.ds(0, B)], buf0, sem0).wait()
    accs = sum_chunk(buf0, accs)

    pltpu.make_async_copy(
        outT_hbm.at[pl.ds(0, 8), pl.ds(0, 128)], zbuf, semg).wait()

    svec = zero
    span0 = pl.multiple_of((b0 & 127) & ~15, 16)
    for wave in range(BPW // 8):
        for k in range(8):
            i = wave * 8 + k
            t = get_t(i)
            tr8 = pl.multiple_of(
                (jnp.minimum(jnp.maximum(t, 0), V - 1)) & ~7, 8)
            pltpu.async_copy(
                outT_hbm.at[pl.ds(tr8, 8), pl.ds(bcol, 128)],
                tbuf.at[k], semg)
        for k in range(8):
            pltpu.make_async_copy(
                outT_hbm.at[pl.ds(0, 8), pl.ds(0, 128)], tbuf.at[k],
                semg).wait()
        for k in range(8):
            i = wave * 8 + k
            t = get_t(i)
            lane = i & 15
            sp = span0 + (i // 16) * 16
            zspan = zbuf[Z_COL & 7, pl.ds(sp, 16)]
            isz = jnp.where(t == Z_COL, jnp.float32(1.0), jnp.float32(0.0))
            zterm = (jnp.float32(SMOOTH) * zspan
                     + isz * (jnp.float32(SLOGS)
                              - jnp.float32(SMOOTH) * zspan))
            svec = svec + jnp.where(iota == lane, zterm, jnp.float32(0.0))
            tv = zero
            for r8 in range(8):
                m = jnp.where((t & 7) == r8, jnp.float32(SMOOTH - CONF),
                              jnp.float32(0.0))
                tv = tv + m * tbuf[k, r8, pl.ds(sp, 16)]
            svec = svec + jnp.where(iota == lane, tv, jnp.float32(0.0))

    total_vec = accs[0]
    for a in accs[1:]:
        total_vec = total_vec + a
    vec = jnp.float32(-SMOOTH) * total_vec + svec
    out_v[...] = vec
    pltpu.sync_copy(out_v, res_hbm.at[pl.ds(wid * 16, 16)])


def _tc_body(x_ref, o_ref):
    i = pl.program_id(0)
    x = x_ref[...]

    @pl.when(i == 0)
    def _():
        o_ref[...] = jnp.zeros_like(o_ref)

    ones = jnp.ones((8, TC_RB), jnp.float32)

    @pl.when(i < TC_NR - 1)
    def _():
        o_ref[...] += jax.lax.dot(ones, x,
                                  preferred_element_type=jnp.float32)

    @pl.when(i == TC_NR - 1)
    def _():
        r0 = (TC_R0 + i) * TC_RB
        rowmat = r0 + lax.broadcasted_iota(jnp.int32, (TC_RB, B), 0)
        xs = jnp.where(rowmat >= V, jnp.float32(0.0), x)
        o_ref[...] += jax.lax.dot(ones, xs,
                                  preferred_element_type=jnp.float32)


@jax.jit
def _loss_parts(output, target):
    out_t = output.T
    mesh = plsc.VectorSubcoreMesh(core_axis_name="c", subcore_axis_name="s")
    sc = pl.kernel(
        _sc_body,
        out_type=jax.ShapeDtypeStruct((NW * 16,), jnp.float32),
        mesh=mesh,
        scratch_types=[
            pltpu.VMEM((CR, B), jnp.float32),
            pltpu.VMEM((CR, B), jnp.float32),
            pltpu.VMEM((8, 8, 128), jnp.float32),
            pltpu.VMEM((8, 128), jnp.float32),
            pltpu.VMEM((BPW,), jnp.int32),
            pltpu.VMEM((16,), jnp.float32),
            pltpu.SemaphoreType.DMA,
            pltpu.SemaphoreType.DMA,
            pltpu.SemaphoreType.DMA,
        ],
    )(out_t, target)

    tc = pl.pallas_call(
        _tc_body,
        out_shape=jax.ShapeDtypeStruct((8, B), jnp.float32),
        grid=(TC_NR,),
        in_specs=[
            pl.BlockSpec((TC_RB, B), lambda i: (i + TC_R0, 0)),
        ],
        out_specs=pl.BlockSpec((8, B), lambda i: (0, 0)),
        compiler_params=pltpu.CompilerParams(
            dimension_semantics=("arbitrary",)),
    )(out_t)
    return sc, tc


def kernel(output, target, norm):
    sc, tc = _loss_parts(output, target)
    total = (jnp.sum(sc) + jnp.float32(-SMOOTH) * jnp.sum(tc)
             + jnp.float32(B * A_CONST))
    return total / jnp.asarray(norm).astype(jnp.float32)


# speedup vs baseline: 1.0216x; 1.0216x over previous
import functools
import math

import jax
import jax.numpy as jnp
from jax import lax
from jax.experimental import pallas as pl
from jax.experimental.pallas import tpu as pltpu
from jax.experimental.pallas import tpu_sc as plsc

B = 1024
V = 100000
SMOOTH = 0.1 / (V - 2)
CONF = 0.9
Z_COL = V - 100
A_CONST = SMOOTH * math.log(SMOOTH) * (V - 2) + CONF * math.log(CONF)
SLOGS = SMOOTH * math.log(SMOOTH)

NW = 32
CR = 32
RPW = 1504
V_SC = NW * RPW
NCH = RPW // CR
BPW = B // NW
UNROLL = 8
NACC = 8
NVEC_ROW = B // 16

TC_RB = 1024
TC_R0 = V_SC // TC_RB
TC_NR = (V - V_SC + TC_RB - 1) // TC_RB


def _sc_body(outT_hbm, tgt_hbm, res_hbm, buf0, buf1, tbuf, zbuf, tgt_v,
             out_v, sem0, sem1, semg):
    wid = lax.axis_index("s") * 2 + lax.axis_index("c")
    v0 = wid * RPW
    b0 = wid * BPW
    pltpu.sync_copy(tgt_hbm.at[pl.ds(b0, BPW)], tgt_v)
    iota = lax.iota(jnp.int32, 16)
    lane0 = jnp.where(iota == 0, jnp.float32(1.0), jnp.float32(0.0))
    zero = jnp.zeros((16,), jnp.float32)

    def get_t(i):
        return tgt_v[pl.ds((i // 16) * 16, 16)][i % 16]

    bcol = pl.multiple_of((b0 & ~127), 128)
    pltpu.async_copy(
        outT_hbm.at[pl.ds((Z_COL // 8) * 8, 8), pl.ds(bcol, 128)], zbuf, semg)

    def dma_chunk(j, buf, sem):
        return pltpu.async_copy(
            outT_hbm.at[pl.ds(v0 + j * CR, CR), pl.ds(0, B)], buf, sem)

    def sum_chunk(buf, accs):
        def row_body(rr, accs):
            def body(ii, accs):
                a = list(accs)
                base = ii * (UNROLL * 16)
                for u in range(UNROLL):
                    a[u] = a[u] + buf[rr, pl.ds(base + u * 16, 16)]
                return tuple(a)
            return lax.fori_loop(0, NVEC_ROW // UNROLL, body, accs)
        return lax.fori_loop(0, CR, row_body, accs)

    dma_chunk(0, buf0, sem0)
    accs = (zero,) * NACC

    def pair_body(k, accs):
        j = 2 * k
        pltpu.make_async_copy(
            outT_hbm.at[pl.ds(0, CR), pl.ds(0, B)], buf0, sem0).wait()
        dma_chunk(j + 1, buf1, sem1)
        accs = sum_chunk(buf0, accs)
        dma_chunk(j + 2, buf0, sem0)
        pltpu.make_async_copy(
            outT_hbm.at[pl.ds(0, CR), pl.ds(0, B)], buf1, sem1).wait()
        accs = sum_chunk(buf1, accs)
        return accs

    accs = lax.fori_loop(0, (NCH - 1) // 2, pair_body, accs)
    pltpu.make_async_copy(
        outT_hbm.at[pl.ds(0, CR), pl.ds(0, B)], buf0, sem0).wait()
    accs = sum_chunk(buf0, accs)

    pltpu.make_async_copy(
        outT_hbm.at[pl.ds(0, 8), pl.ds(0, 128)], zbuf, semg).wait()

    svec = zero
    span0 = pl.multiple_of((b0 & 127) & ~15, 16)
    for wave in range(BPW // 8):
        for k in range(8):
            i = wave * 8 + k
            t = get_t(i)
            tr8 = pl.multiple_of(
                (jnp.minimum(jnp.maximum(t, 0), V - 1)) & ~7, 8)
            pltpu.async_copy(
                outT_hbm.at[pl.ds(tr8, 8), pl.ds(bcol, 128)],
                tbuf.at[k], semg)
        for k in range(8):
            pltpu.make_async_copy(
                outT_hbm.at[pl.ds(0, 8), pl.ds(0, 128)], tbuf.at[k],
                semg).wait()
        for k in range(8):
            i = wave * 8 + k
            t = get_t(i)
            lane = i & 15
            sp = span0 + (i // 16) * 16
            zspan = zbuf[Z_COL & 7, pl.ds(sp, 16)]
            isz = jnp.where(t == Z_COL, jnp.float32(1.0), jnp.float32(0.0))
            zterm = (jnp.float32(SMOOTH) * zspan
                     + isz * (jnp.float32(SLOGS)
                              - jnp.float32(SMOOTH) * zspan))
            svec = svec + jnp.where(iota == lane, zterm, jnp.float32(0.0))
            tv = zero
            for r8 in range(8):
                m = jnp.where((t & 7) == r8, jnp.float32(SMOOTH - CONF),
                              jnp.float32(0.0))
                tv = tv + m * tbuf[k, r8, pl.ds(sp, 16)]
            svec = svec + jnp.where(iota == lane, tv, jnp.float32(0.0))

    total_vec = accs[0]
    for a in accs[1:]:
        total_vec = total_vec + a
    vec = jnp.float32(-SMOOTH) * total_vec + svec
    out_v[...] = vec
    pltpu.sync_copy(out_v, res_hbm.at[pl.ds(wid * 16, 16)])


def _tc_body(x_ref, o_ref):
    i = pl.program_id(0)
    x = x_ref[...]

    @pl.when(i == 0)
    def _():
        o_ref[...] = jnp.zeros_like(o_ref)

    @pl.when(i < TC_NR - 1)
    def _():
        o_ref[...] += x.reshape(TC_RB // 8, 8, B // 128, 128).sum(axis=(0, 2))

    @pl.when(i == TC_NR - 1)
    def _():
        r0 = (TC_R0 + i) * TC_RB
        rowmat = r0 + lax.broadcasted_iota(jnp.int32, (TC_RB, B), 0)
        xs = jnp.where(rowmat >= V, jnp.float32(0.0), x)
        o_ref[...] += xs.reshape(TC_RB // 8, 8, B // 128, 128).sum(axis=(0, 2))


@jax.jit
def _loss_parts(output, target):
    out_t = output.T
    mesh = plsc.VectorSubcoreMesh(core_axis_name="c", subcore_axis_name="s")
    sc = pl.kernel(
        _sc_body,
        out_type=jax.ShapeDtypeStruct((NW * 16,), jnp.float32),
        mesh=mesh,
        scratch_types=[
            pltpu.VMEM((CR, B), jnp.float32),
            pltpu.VMEM((CR, B), jnp.float32),
            pltpu.VMEM((8, 8, 128), jnp.float32),
            pltpu.VMEM((8, 128), jnp.float32),
            pltpu.VMEM((BPW,), jnp.int32),
            pltpu.VMEM((16,), jnp.float32),
            pltpu.SemaphoreType.DMA,
            pltpu.SemaphoreType.DMA,
            pltpu.SemaphoreType.DMA,
        ],
    )(out_t, target)

    tc = pl.pallas_call(
        _tc_body,
        out_shape=jax.ShapeDtypeStruct((8, 128), jnp.float32),
        grid=(TC_NR,),
        in_specs=[
            pl.BlockSpec((TC_RB, B), lambda i: (i + TC_R0, 0)),
        ],
        out_specs=pl.BlockSpec((8, 128), lambda i: (0, 0)),
        compiler_params=pltpu.CompilerParams(
            dimension_semantics=("arbitrary",)),
    )(out_t)
    return sc, tc


def kernel(output, target, norm):
    sc, tc = _loss_parts(output, target)
    total = (jnp.sum(sc) + jnp.float32(-SMOOTH) * jnp.sum(tc)
             + jnp.float32(B * A_CONST))
    return total / jnp.asarray(norm).astype(jnp.float32)
